# baseline (speedup 1.0000x reference)
import functools

import jax
import jax.numpy as jnp
from jax import lax
from jax.experimental import pallas as pl
from jax.experimental.pallas import tpu as pltpu
from jax.experimental.pallas import tpu_sc as plsc

B, H, S, D = 2, 16, 4096, 64
W = B * H
L = 16
NV = S // L
NC = 2
RB = (W * D) // 8
CB = S // 128
FIX_SWEEPS = 3


def _rowsum_body(vt_ref, s_ref):
    xt = vt_ref[0]
    acc = xt[0:8, :]
    for g in range(1, 8):
        acc = acc + xt[8 * g:8 * g + 8, :]
    a = acc[0:4, :] + acc[4:8, :]
    b = a[0:2, :] + a[2:4, :]
    s = b[0, :] + b[1, :]
    s_ref[...] = s.reshape(CB, 128)[None]


def _rowsums(vt):
    return pl.pallas_call(
        _rowsum_body,
        grid=(W,),
        in_specs=[pl.BlockSpec((1, D, S), lambda i: (i, 0, 0))],
        out_specs=pl.BlockSpec((1, CB, 128), lambda i: (i, 0, 0)),
        out_shape=jax.ShapeDtypeStruct((W, CB, 128), jnp.float32),
    )(vt)


@functools.partial(
    pl.kernel,
    out_type=jax.ShapeDtypeStruct((RB, CB, 8, 128), jnp.float32),
    mesh=plsc.VectorSubcoreMesh(core_axis_name="c", subcore_axis_name="s"),
    compiler_params=pltpu.CompilerParams(
        needs_layout_passes=False, use_tc_tiling_on_sc=False
    ),
    scratch_types=[
        pltpu.VMEM((CB, 128), jnp.float32),
        pltpu.VMEM((S,), jnp.float32),
        pltpu.VMEM((S,), jnp.int32),
        pltpu.VMEM((S,), jnp.float32),
        pltpu.VMEM((S,), jnp.int32),
        pltpu.VMEM((CB, 8, 128), jnp.float32),
        pltpu.VMEM((CB, 8, 128), jnp.float32),
        pltpu.VMEM((CB, 8, 128), jnp.float32),
        pltpu.SemaphoreType.DMA,
        pltpu.SemaphoreType.DMA,
        pltpu.SemaphoreType.DMA,
    ],
)
def _sc_sort_permute(sums_hbm, vt_hbm, out_hbm, kst, ka, va, kb, vb, vin0,
                     vin1, vout, sem0, sem1, osem):
    wid = lax.axis_index("s") * NC + lax.axis_index("c")
    iota = lax.iota(jnp.int32, L)

    pltpu.sync_copy(sums_hbm.at[wid], kst)

    @plsc.parallel_loop(0, NV, unroll=2)
    def _repack(i):
        ka[pl.ds(i * L, L)] = kst[i // 8, pl.ds((i - (i // 8) * 8) * L, L)]
        va[pl.ds(i * L, L)] = iota + i * L

    @plsc.parallel_loop(0, NV, unroll=2)
    def _run16(i):
        o = i * L
        k, v = plsc.sort_key_val(ka[pl.ds(o, L)], va[pl.ds(o, L)])
        ka[pl.ds(o, L)] = k
        va[pl.ds(o, L)] = v

    for m in (1, 2, 4, 8):
        span = 2 * m * L

        @plsc.parallel_loop(0, NV // (2 * m), unroll=2 if m <= 4 else 1)
        def _pairf(p, m=m, span=span):
            base = p * span
            ks = [ka[pl.ds(base + j * L, L)] for j in range(m)]
            vs = [va[pl.ds(base + j * L, L)] for j in range(m)]
            for j in range(m):
                srco = base + (2 * m - 1 - j) * L
                ks.append(lax.rev(ka[pl.ds(srco, L)], (0,)))
                vs.append(lax.rev(va[pl.ds(srco, L)], (0,)))
            d = m
            while d >= 1:
                for t in range(m):
                    blk = t // d
                    i = t - blk * d
                    p1 = blk * 2 * d + i
                    p2 = p1 + d
                    cle = ks[p1] <= ks[p2]
                    klo = jnp.where(cle, ks[p1], ks[p2])
                    khi = jnp.where(cle, ks[p2], ks[p1])
                    vlo = jnp.where(cle, vs[p1], vs[p2])
                    vhi = jnp.where(cle, vs[p2], vs[p1])
                    ks[p1], ks[p2] = klo, khi
                    vs[p1], vs[p2] = vlo, vhi
                d //= 2
            for j in range(2 * m):
                k2, v2 = plsc.sort_key_val(ks[j], vs[j])
                ka[pl.ds(base + j * L, L)] = k2
                va[pl.ds(base + j * L, L)] = v2

    m = 16
    while m < NV:
        span = 2 * m * L

        def _pair(p, c, m=m, span=span):
            base = p * span

            @plsc.parallel_loop(0, m, unroll=2)
            def _stage0(t, m=m, base=base):
                xk = ka[pl.ds(base + t * L, L)]
                xv = va[pl.ds(base + t * L, L)]
                srco = base + (2 * m - 1 - t) * L
                yk = lax.rev(ka[pl.ds(srco, L)], (0,))
                yv = lax.rev(va[pl.ds(srco, L)], (0,))
                cle = xk <= yk
                kb[pl.ds(base + t * L, L)] = jnp.where(cle, xk, yk)
                kb[pl.ds(base + (t + m) * L, L)] = jnp.where(cle, yk, xk)
                vb[pl.ds(base + t * L, L)] = jnp.where(cle, xv, yv)
                vb[pl.ds(base + (t + m) * L, L)] = jnp.where(cle, yv, xv)

            d = m // 2
            while d >= 1:
                @plsc.parallel_loop(0, m, unroll=2)
                def _stage(t, d=d, base=base):
                    blk = t // d
                    i = t - blk * d
                    p1 = base + (blk * 2 * d + i) * L
                    p2 = p1 + d * L
                    xk = kb[pl.ds(p1, L)]
                    yk = kb[pl.ds(p2, L)]
                    xv = vb[pl.ds(p1, L)]
                    yv = vb[pl.ds(p2, L)]
                    cle = xk <= yk
                    kb[pl.ds(p1, L)] = jnp.where(cle, xk, yk)
                    kb[pl.ds(p2, L)] = jnp.where(cle, yk, xk)
                    vb[pl.ds(p1, L)] = jnp.where(cle, xv, yv)
                    vb[pl.ds(p2, L)] = jnp.where(cle, yv, xv)
                d //= 2

            @plsc.parallel_loop(0, 2 * m, unroll=2)
            def _fin(j, base=base):
                o = base + j * L
                k, v = plsc.sort_key_val(kb[pl.ds(o, L)], vb[pl.ds(o, L)])
                ka[pl.ds(o, L)] = k
                va[pl.ds(o, L)] = v
            return c
        lax.fori_loop(0, NV // (2 * m), _pair, None)
        m *= 2

    even_idx = iota * 2
    for _ in range(FIX_SWEEPS):
        for ph in (0, 1):
            @plsc.parallel_loop(0, S // (2 * L), unroll=2)
            def _fix(t, ph=ph):
                i1 = t * (2 * L) + even_idx + ph
                i2 = jnp.minimum(i1 + 1, S - 1)
                k1 = plsc.load_gather(ka, [i1])
                k2 = plsc.load_gather(ka, [i2])
                v1 = plsc.load_gather(va, [i1])
                v2 = plsc.load_gather(va, [i2])
                sw = (k1 == k2) & (v1 > v2)
                plsc.store_scatter(va, [i1], jnp.where(sw, v2, v1))
                plsc.store_scatter(va, [i2], jnp.where(sw, v1, v2))

    nblk = D // 8
    bufs = (vin0, vin1)
    sems = (sem0, sem1)
    rb0 = wid * nblk
    pltpu.async_copy(vt_hbm.at[rb0], vin0, sem0)
    for t in range(nblk):
        cur, csem = bufs[t % 2], sems[t % 2]
        if t + 1 < nblk:
            pltpu.async_copy(
                vt_hbm.at[rb0 + t + 1], bufs[(t + 1) % 2], sems[(t + 1) % 2])
        pltpu.make_async_copy(vt_hbm.at[rb0 + t], cur, csem).wait()
        if t > 0:
            for q in range(4):
                pltpu.make_async_copy(
                    vout.at[pl.ds(q * 8, 8)],
                    out_hbm.at[rb0 + t - 1, pl.ds(q * 8, 8)], osem).wait()
        for q in range(4):
            @plsc.parallel_loop(q * 64, (q + 1) * 64, unroll=2)
            def _perm_vec(i, cur=cur):
                idxv = va[pl.ds(i * L, L)]
                hi = lax.shift_right_logical(idxv, 7)
                lo = lax.bitwise_and(idxv, 127)
                pb = i // 8
                po = (i - pb * 8) * L
                for r in range(8):
                    g = plsc.load_gather(cur, [hi, iota * 0 + r, lo])
                    vout[pb, r, pl.ds(po, L)] = g
            pltpu.async_copy(
                vout.at[pl.ds(q * 8, 8)],
                out_hbm.at[rb0 + t, pl.ds(q * 8, 8)], osem)
    for q in range(4):
        pltpu.make_async_copy(
            vout.at[pl.ds(q * 8, 8)],
            out_hbm.at[rb0 + nblk - 1, pl.ds(q * 8, 8)], osem).wait()


def kernel(q, k, v):
    del q, k
    vt = jnp.transpose(v, (0, 1, 3, 2)).reshape(W, D, S)
    sums = _rowsums(vt)
    v_tiles = (
        vt.reshape(RB, 8, CB, 128).transpose(0, 2, 1, 3)
    )
    out_tiles = _sc_sort_permute(sums, v_tiles)
    out_t = out_tiles.transpose(0, 2, 1, 3).reshape(B, H, D, S)
    out = jnp.transpose(out_t, (0, 1, 3, 2))
    return (out, out)


# speedup vs baseline: 1.0079x; 1.0079x over previous
import functools

import jax
import jax.numpy as jnp
from jax import lax
from jax.experimental import pallas as pl
from jax.experimental.pallas import tpu as pltpu
from jax.experimental.pallas import tpu_sc as plsc

B, H, S, D = 2, 16, 4096, 64
W = B * H
L = 16
NV = S // L
NC = 2
RB = (W * D) // 8
CB = S // 128
FIX_SWEEPS = 3


def _rowsum_body(vt_ref, s_ref):
    xt = vt_ref[0]
    acc = xt[0:8, :]
    for g in range(1, 8):
        acc = acc + xt[8 * g:8 * g + 8, :]
    a = acc[0:4, :] + acc[4:8, :]
    b = a[0:2, :] + a[2:4, :]
    s = b[0, :] + b[1, :]
    s_ref[...] = s.reshape(CB, 128)[None]


def _rowsums(vt):
    return pl.pallas_call(
        _rowsum_body,
        grid=(W,),
        in_specs=[pl.BlockSpec((1, D, S), lambda i: (i, 0, 0))],
        out_specs=pl.BlockSpec((1, CB, 128), lambda i: (i, 0, 0)),
        out_shape=jax.ShapeDtypeStruct((W, CB, 128), jnp.float32),
    )(vt)


@functools.partial(
    pl.kernel,
    out_type=jax.ShapeDtypeStruct((RB, CB, 8, 128), jnp.float32),
    mesh=plsc.VectorSubcoreMesh(core_axis_name="c", subcore_axis_name="s"),
    compiler_params=pltpu.CompilerParams(
        needs_layout_passes=False, use_tc_tiling_on_sc=False
    ),
    scratch_types=[
        pltpu.VMEM((CB, 128), jnp.float32),
        pltpu.VMEM((S,), jnp.float32),
        pltpu.VMEM((S,), jnp.int32),
        pltpu.VMEM((S,), jnp.float32),
        pltpu.VMEM((S,), jnp.int32),
        pltpu.VMEM((CB, 8, 128), jnp.float32),
        pltpu.VMEM((CB, 8, 128), jnp.float32),
        pltpu.VMEM((CB, 8, 128), jnp.float32),
        pltpu.SemaphoreType.DMA,
        pltpu.SemaphoreType.DMA,
    ],
)
def _sc_sort_permute(sums_hbm, vt_hbm, out_hbm, kst, ka, va, kb, vb, vin0,
                     vin1, vout, sem0, sem1):
    wid = lax.axis_index("s") * NC + lax.axis_index("c")
    iota = lax.iota(jnp.int32, L)

    pltpu.sync_copy(sums_hbm.at[wid], kst)

    @plsc.parallel_loop(0, NV, unroll=2)
    def _repack(i):
        ka[pl.ds(i * L, L)] = kst[i // 8, pl.ds((i - (i // 8) * 8) * L, L)]
        va[pl.ds(i * L, L)] = iota + i * L

    @plsc.parallel_loop(0, NV, unroll=2)
    def _run16(i):
        o = i * L
        k, v = plsc.sort_key_val(ka[pl.ds(o, L)], va[pl.ds(o, L)])
        ka[pl.ds(o, L)] = k
        va[pl.ds(o, L)] = v

    for m in (1, 2, 4, 8):
        span = 2 * m * L

        @plsc.parallel_loop(0, NV // (2 * m), unroll=2 if m <= 4 else 1)
        def _pairf(p, m=m, span=span):
            base = p * span
            ks = [ka[pl.ds(base + j * L, L)] for j in range(m)]
            vs = [va[pl.ds(base + j * L, L)] for j in range(m)]
            for j in range(m):
                srco = base + (2 * m - 1 - j) * L
                ks.append(lax.rev(ka[pl.ds(srco, L)], (0,)))
                vs.append(lax.rev(va[pl.ds(srco, L)], (0,)))
            d = m
            while d >= 1:
                for t in range(m):
                    blk = t // d
                    i = t - blk * d
                    p1 = blk * 2 * d + i
                    p2 = p1 + d
                    cle = ks[p1] <= ks[p2]
                    klo = jnp.where(cle, ks[p1], ks[p2])
                    khi = jnp.where(cle, ks[p2], ks[p1])
                    vlo = jnp.where(cle, vs[p1], vs[p2])
                    vhi = jnp.where(cle, vs[p2], vs[p1])
                    ks[p1], ks[p2] = klo, khi
                    vs[p1], vs[p2] = vlo, vhi
                d //= 2
            for j in range(2 * m):
                k2, v2 = plsc.sort_key_val(ks[j], vs[j])
                ka[pl.ds(base + j * L, L)] = k2
                va[pl.ds(base + j * L, L)] = v2

    m = 16
    while m < NV:
        span = 2 * m * L

        def _pair(p, c, m=m, span=span):
            base = p * span

            @plsc.parallel_loop(0, m, unroll=2)
            def _stage0(t, m=m, base=base):
                xk = ka[pl.ds(base + t * L, L)]
                xv = va[pl.ds(base + t * L, L)]
                srco = base + (2 * m - 1 - t) * L
                yk = lax.rev(ka[pl.ds(srco, L)], (0,))
                yv = lax.rev(va[pl.ds(srco, L)], (0,))
                cle = xk <= yk
                kb[pl.ds(base + t * L, L)] = jnp.where(cle, xk, yk)
                kb[pl.ds(base + (t + m) * L, L)] = jnp.where(cle, yk, xk)
                vb[pl.ds(base + t * L, L)] = jnp.where(cle, xv, yv)
                vb[pl.ds(base + (t + m) * L, L)] = jnp.where(cle, yv, xv)

            d = m // 2
            while d >= 1:
                @plsc.parallel_loop(0, m, unroll=2)
                def _stage(t, d=d, base=base):
                    blk = t // d
                    i = t - blk * d
                    p1 = base + (blk * 2 * d + i) * L
                    p2 = p1 + d * L
                    xk = kb[pl.ds(p1, L)]
                    yk = kb[pl.ds(p2, L)]
                    xv = vb[pl.ds(p1, L)]
                    yv = vb[pl.ds(p2, L)]
                    cle = xk <= yk
                    kb[pl.ds(p1, L)] = jnp.where(cle, xk, yk)
                    kb[pl.ds(p2, L)] = jnp.where(cle, yk, xk)
                    vb[pl.ds(p1, L)] = jnp.where(cle, xv, yv)
                    vb[pl.ds(p2, L)] = jnp.where(cle, yv, xv)
                d //= 2

            @plsc.parallel_loop(0, 2 * m, unroll=2)
            def _fin(j, base=base):
                o = base + j * L
                k, v = plsc.sort_key_val(kb[pl.ds(o, L)], vb[pl.ds(o, L)])
                ka[pl.ds(o, L)] = k
                va[pl.ds(o, L)] = v
            return c
        lax.fori_loop(0, NV // (2 * m), _pair, None)
        m *= 2

    even_idx = iota * 2
    for _ in range(FIX_SWEEPS):
        for ph in (0, 1):
            @plsc.parallel_loop(0, S // (2 * L), unroll=2)
            def _fix(t, ph=ph):
                i1 = t * (2 * L) + even_idx + ph
                i2 = jnp.minimum(i1 + 1, S - 1)
                k1 = plsc.load_gather(ka, [i1])
                k2 = plsc.load_gather(ka, [i2])
                v1 = plsc.load_gather(va, [i1])
                v2 = plsc.load_gather(va, [i2])
                sw = (k1 == k2) & (v1 > v2)
                plsc.store_scatter(va, [i1], jnp.where(sw, v2, v1))
                plsc.store_scatter(va, [i2], jnp.where(sw, v1, v2))

    nblk = D // 8
    bufs = (vin0, vin1)
    sems = (sem0, sem1)
    rb0 = wid * nblk
    pltpu.async_copy(vt_hbm.at[rb0], vin0, sem0)
    for t in range(nblk):
        cur, csem = bufs[t % 2], sems[t % 2]
        if t + 1 < nblk:
            pltpu.async_copy(
                vt_hbm.at[rb0 + t + 1], bufs[(t + 1) % 2], sems[(t + 1) % 2])
        pltpu.make_async_copy(vt_hbm.at[rb0 + t], cur, csem).wait()

        @plsc.parallel_loop(0, NV, unroll=2)
        def _perm_vec(i, cur=cur):
            idxv = va[pl.ds(i * L, L)]
            hi = lax.shift_right_logical(idxv, 7)
            lo = lax.bitwise_and(idxv, 127)
            pb = i // 8
            po = (i - pb * 8) * L
            for r in range(8):
                g = plsc.load_gather(cur, [hi, iota * 0 + r, lo])
                vout[pb, r, pl.ds(po, L)] = g
        pltpu.sync_copy(vout, out_hbm.at[rb0 + t])


def kernel(q, k, v):
    del q, k
    vt = jnp.transpose(v, (0, 1, 3, 2)).reshape(W, D, S)
    sums = _rowsums(vt)
    v_tiles = (
        vt.reshape(RB, 8, CB, 128).transpose(0, 2, 1, 3)
    )
    out_tiles = _sc_sort_permute(sums, v_tiles)
    out_t = out_tiles.transpose(0, 2, 1, 3).reshape(B, H, D, S)
    out = jnp.transpose(out_t, (0, 1, 3, 2))
    return (out, out)
